# Initial kernel scaffold; baseline (speedup 1.0000x reference)
import jax
import jax.numpy as jnp
from jax.experimental import pallas as pl


def kernel(query, value, reference_points, spatial_shapes, W_so, b_so, W_aw, b_aw, W_v, b_v, W_o, b_o):
    raise NotImplementedError("write your pallas kernel here")


# speedup vs baseline: 640.7755x; 640.7755x over previous
import functools

import jax
import jax.numpy as jnp
import numpy as np
from jax import lax
from jax.experimental import pallas as pl
from jax.experimental.pallas import tpu as pltpu
from jax.experimental.pallas import tpu_sc as plsc

EMBED = 256
HEADS = 8
QUEUE = 2
POINTS = 4
H0 = 150
W0 = 150
NQ = H0 * W0
HEAD_DIM = EMBED // HEADS
NQP = 22528
NTAP = QUEUE * HEADS * POINTS * 4
K64 = QUEUE * HEADS * POINTS

def _perm_so(xy):
    rows = []
    for q in range(QUEUE):
        for h in range(HEADS):
            for p in range(POINTS):
                rows.append(((h * QUEUE + q) * POINTS + p) * 2 + xy)
    return np.array(rows, dtype=np.int32)

def _perm_aw():
    rows = []
    for q in range(QUEUE):
        for h in range(HEADS):
            for p in range(POINTS):
                rows.append((h * QUEUE + q) * POINTS + p)
    return np.array(rows, dtype=np.int32)

_PX = _perm_so(0)
_PY = _perm_so(1)
_PA = _perm_aw()
_M4 = np.kron(np.eye(K64 // POINTS, dtype=np.float32), np.ones((POINTS, POINTS), np.float32))
_R1 = np.zeros((K64, K64), np.float32)
_R2 = np.zeros((K64, K64), np.float32)
for _i in range(K64):
    _R1[_i, _i ^ 1] = 1.0
    _R2[_i, _i ^ 2] = 1.0
_QB = (np.arange(K64, dtype=np.float32) // POINTS) * NQP


def _a1_body(x_ref, wt_ref, b_ref, out_ref):
    x = x_ref[0]
    out_ref[0] = jnp.dot(x, wt_ref[0], preferred_element_type=jnp.float32) + b_ref[0]


def _a1(value_p, W_v, b_v, tq=512):
    nt = NQP // tq
    wvt = W_v.T.reshape(1, EMBED, HEADS, HEAD_DIM)
    wvt = jnp.transpose(wvt, (2, 0, 1, 3)).reshape(HEADS, EMBED, HEAD_DIM)
    bv = b_v.reshape(HEADS, 1, HEAD_DIM)
    return pl.pallas_call(
        _a1_body,
        grid=(QUEUE, HEADS, nt),
        in_specs=[
            pl.BlockSpec((1, tq, EMBED), lambda q, h, t: (q, t, 0)),
            pl.BlockSpec((1, EMBED, HEAD_DIM), lambda q, h, t: (h, 0, 0)),
            pl.BlockSpec((1, 1, HEAD_DIM), lambda q, h, t: (h, 0, 0)),
        ],
        out_specs=pl.BlockSpec((1, tq, HEAD_DIM), lambda q, h, t: (q * HEADS + h, t, 0)),
        out_shape=jax.ShapeDtypeStruct((QUEUE * HEADS, NQP, HEAD_DIM), jnp.float32),
    )(value_p, wvt.reshape(HEADS, EMBED, HEAD_DIM), bv)


def _a2_body(qc_ref, wx_ref, bx_ref, wy_ref, by_ref, wa_ref, ba_ref,
             ref_ref, m4_ref, r1_ref, r2_ref, qb_ref, idx_ref, w_ref):
    qc = qc_ref[...]
    x64 = jnp.dot(qc, wx_ref[...], preferred_element_type=jnp.float32) + bx_ref[...]
    y64 = jnp.dot(qc, wy_ref[...], preferred_element_type=jnp.float32) + by_ref[...]
    a64 = jnp.dot(qc, wa_ref[...], preferred_element_type=jnp.float32) + ba_ref[...]
    m = jnp.maximum(a64, jnp.dot(a64, r1_ref[...], preferred_element_type=jnp.float32))
    m = jnp.maximum(m, jnp.dot(m, r2_ref[...], preferred_element_type=jnp.float32))
    e = jnp.exp(a64 - m)
    s = jnp.dot(e, m4_ref[...], preferred_element_type=jnp.float32)
    aw = e / s

    rx = ref_ref[:, 0:1]
    ry = ref_ref[:, 1:2]
    x = rx * W0 + x64 - 0.5
    y = ry * H0 + y64 - 0.5
    x0 = jnp.floor(x)
    y0 = jnp.floor(y)
    fx = x - x0
    fy = y - y0
    x1 = x0 + 1.0
    y1 = y0 + 1.0

    def corner(cx, cy, wgt):
        valid = ((cx >= 0.0) & (cx <= W0 - 1.0) & (cy >= 0.0) & (cy <= H0 - 1.0))
        cxc = jnp.clip(cx, 0.0, W0 - 1.0)
        cyc = jnp.clip(cy, 0.0, H0 - 1.0)
        flat = cyc * W0 + cxc + qb_ref[...]
        wv = wgt * aw * valid.astype(jnp.float32)
        return flat, wv

    f0, w0 = corner(x0, y0, (1.0 - fx) * (1.0 - fy))
    f1, w1 = corner(x0, y1, (1.0 - fx) * fy)
    f2, w2 = corner(x1, y0, fx * (1.0 - fy))
    f3, w3 = corner(x1, y1, fx * fy)
    idx_f = jnp.concatenate([f0, f1, f2, f3], axis=1)
    idx_ref[...] = idx_f.astype(jnp.int32)
    w_ref[...] = jnp.concatenate([w0, w1, w2, w3], axis=1)


def _a2(qc, wxt, bx, wyt, by, wat, ba, refp, tq=512):
    nt = NQP // tq
    consts = dict(
        m4=jnp.asarray(_M4), r1=jnp.asarray(_R1), r2=jnp.asarray(_R2),
        qb=jnp.asarray(_QB).reshape(1, K64),
    )
    full = lambda shape: pl.BlockSpec(shape, lambda t: tuple(0 for _ in shape))
    return pl.pallas_call(
        _a2_body,
        grid=(nt,),
        in_specs=[
            pl.BlockSpec((tq, QUEUE * EMBED), lambda t: (t, 0)),
            full((QUEUE * EMBED, K64)), full((1, K64)),
            full((QUEUE * EMBED, K64)), full((1, K64)),
            full((QUEUE * EMBED, K64)), full((1, K64)),
            pl.BlockSpec((tq, 2), lambda t: (t, 0)),
            full((K64, K64)), full((K64, K64)), full((K64, K64)), full((1, K64)),
        ],
        out_specs=[
            pl.BlockSpec((tq, NTAP), lambda t: (t, 0)),
            pl.BlockSpec((tq, NTAP), lambda t: (t, 0)),
        ],
        out_shape=[
            jax.ShapeDtypeStruct((NQP, NTAP), jnp.int32),
            jax.ShapeDtypeStruct((NQP, NTAP), jnp.float32),
        ],
    )(qc, wxt, bx.reshape(1, K64), wyt, by.reshape(1, K64), wat,
      ba.reshape(1, K64), refp, consts["m4"], consts["r1"], consts["r2"], consts["qb"])


def _sc_gather(table, idx):
    B = idx.shape[0]
    info = plsc.get_sparse_core_info()
    nw = info.num_cores * info.num_subcores
    bw = B // nw
    chunk = 2048
    while bw % chunk:
        chunk //= 2
    niter = bw // chunk
    mesh = plsc.VectorSubcoreMesh(core_axis_name="c", subcore_axis_name="s")

    @functools.partial(
        pl.kernel, mesh=mesh,
        compiler_params=pltpu.CompilerParams(use_tc_tiling_on_sc=False),
        out_type=jax.ShapeDtypeStruct((B, HEAD_DIM), jnp.float32),
        scratch_types=[
            pltpu.VMEM((chunk,), jnp.int32),
            pltpu.VMEM((chunk, HEAD_DIM), jnp.float32),
            pltpu.SemaphoreType.DMA,
        ],
    )
    def k(table_hbm, idx_hbm, out_hbm, idx_v, rows_v, sem):
        wid = lax.axis_index("s") * info.num_cores + lax.axis_index("c")
        base0 = wid * bw

        def body(j, carry):
            base = base0 + j * chunk
            pltpu.sync_copy(idx_hbm.at[pl.ds(base, chunk)], idx_v)
            pltpu.async_copy(table_hbm.at[idx_v], rows_v, sem).wait()
            pltpu.sync_copy(rows_v, out_hbm.at[pl.ds(base, chunk)])
            return carry

        lax.fori_loop(0, niter, body, 0)

    return k(table, idx)


def _c_body(g_ref, w_ref, q_ref, wo_ref, bo_ref, out_ref):
    g = g_ref[...]
    w = w_ref[...]
    accs = [None] * (QUEUE * HEADS)
    for t in range(NTAP):
        k = t % K64
        qh = k // POINTS
        term = g[:, t * HEAD_DIM:(t + 1) * HEAD_DIM] * w[:, t:t + 1]
        accs[qh] = term if accs[qh] is None else accs[qh] + term
    o = jnp.concatenate(accs, axis=1)
    mean = 0.5 * (o[:, :EMBED] + o[:, EMBED:])
    out_ref[...] = (jnp.dot(mean, wo_ref[...], preferred_element_type=jnp.float32)
                    + bo_ref[...] + q_ref[...])


def _c(g2d, w2d, query_p, W_o, b_o, tq=128):
    nt = NQP // tq
    return pl.pallas_call(
        _c_body,
        grid=(nt,),
        in_specs=[
            pl.BlockSpec((tq, NTAP * HEAD_DIM), lambda t: (t, 0)),
            pl.BlockSpec((tq, NTAP), lambda t: (t, 0)),
            pl.BlockSpec((tq, EMBED), lambda t: (t, 0)),
            pl.BlockSpec((EMBED, EMBED), lambda t: (0, 0)),
            pl.BlockSpec((1, EMBED), lambda t: (0, 0)),
        ],
        out_specs=pl.BlockSpec((tq, EMBED), lambda t: (t, 0)),
        out_shape=jax.ShapeDtypeStruct((NQP, EMBED), jnp.float32),
    )(g2d, w2d, query_p, W_o.T, b_o.reshape(1, EMBED))


def kernel(query, value, reference_points, spatial_shapes, W_so, b_so, W_aw, b_aw, W_v, b_v, W_o, b_o):
    pad = NQP - NQ
    value2 = value.reshape(QUEUE, NQ, EMBED)
    value_p = jnp.pad(value2, ((0, 0), (0, pad), (0, 0)))
    query_p = jnp.pad(query[0], ((0, pad), (0, 0)))
    refp = jnp.pad(reference_points.reshape(NQ, 2), ((0, pad), (0, 0)))
    qc = jnp.concatenate([value_p[0], query_p], axis=-1)

    table = _a1(value_p, W_v, b_v)
    wxt = W_so[jnp.asarray(_PX)].T
    wyt = W_so[jnp.asarray(_PY)].T
    bx = b_so[jnp.asarray(_PX)]
    by = b_so[jnp.asarray(_PY)]
    wat = W_aw[jnp.asarray(_PA)].T
    ba = b_aw[jnp.asarray(_PA)]
    idx, w = _a2(qc, wxt, bx, wyt, by, wat, ba, refp)

    g = _sc_gather(table.reshape(QUEUE * HEADS * NQP, HEAD_DIM),
                   idx.reshape(NQP * NTAP))
    out_p = _c(g.reshape(NQP, NTAP * HEAD_DIM), w, query_p, W_o, b_o)
    return out_p[None, :NQ, :]


# speedup vs baseline: 1318.3093x; 2.0574x over previous
import functools

import jax
import jax.numpy as jnp
import numpy as np
from jax import lax
from jax.experimental import pallas as pl
from jax.experimental.pallas import tpu as pltpu
from jax.experimental.pallas import tpu_sc as plsc

EMBED = 256
HEADS = 8
QUEUE = 2
POINTS = 4
H0 = 150
W0 = 150
NQ = H0 * W0
HEAD_DIM = EMBED // HEADS
NQP = 22528
NTAP = QUEUE * HEADS * POINTS * 4
K64 = QUEUE * HEADS * POINTS

def _perm_so(xy):
    rows = []
    for p in range(POINTS):
        for q in range(QUEUE):
            for h in range(HEADS):
                rows.append(((h * QUEUE + q) * POINTS + p) * 2 + xy)
    return np.array(rows, dtype=np.int32)

def _perm_aw():
    rows = []
    for p in range(POINTS):
        for q in range(QUEUE):
            for h in range(HEADS):
                rows.append((h * QUEUE + q) * POINTS + p)
    return np.array(rows, dtype=np.int32)

_PX = _perm_so(0)
_PY = _perm_so(1)
_PA = _perm_aw()
_NQH = QUEUE * HEADS
_M4 = np.zeros((K64, K64), np.float32)
_R1 = np.zeros((K64, K64), np.float32)
_R2 = np.zeros((K64, K64), np.float32)
for _i in range(K64):
    for _j in range(K64):
        if _i % _NQH == _j % _NQH:
            _M4[_i, _j] = 1.0
    _R1[_i, _i ^ 16] = 1.0
    _R2[_i, _i ^ 32] = 1.0
_QB = (np.arange(K64, dtype=np.float32) % _NQH) * NQP
_REP = np.zeros((_NQH, _NQH * HEAD_DIM), np.float32)
for _i in range(_NQH):
    _REP[_i, _i * HEAD_DIM:(_i + 1) * HEAD_DIM] = 1.0


def _a1_body(x_ref, wt_ref, b_ref, out_ref):
    x = x_ref[0]
    out_ref[0] = jnp.dot(x, wt_ref[0], preferred_element_type=jnp.float32) + b_ref[0]


def _a1(value_p, W_v, b_v, tq=512):
    nt = NQP // tq
    wvt = W_v.T.reshape(1, EMBED, HEADS, HEAD_DIM)
    wvt = jnp.transpose(wvt, (2, 0, 1, 3)).reshape(HEADS, EMBED, HEAD_DIM)
    bv = b_v.reshape(HEADS, 1, HEAD_DIM)
    return pl.pallas_call(
        _a1_body,
        grid=(QUEUE, HEADS, nt),
        in_specs=[
            pl.BlockSpec((1, tq, EMBED), lambda q, h, t: (q, t, 0)),
            pl.BlockSpec((1, EMBED, HEAD_DIM), lambda q, h, t: (h, 0, 0)),
            pl.BlockSpec((1, 1, HEAD_DIM), lambda q, h, t: (h, 0, 0)),
        ],
        out_specs=pl.BlockSpec((1, tq, HEAD_DIM), lambda q, h, t: (q * HEADS + h, t, 0)),
        out_shape=jax.ShapeDtypeStruct((QUEUE * HEADS, NQP, HEAD_DIM), jnp.float32),
    )(value_p, wvt.reshape(HEADS, EMBED, HEAD_DIM), bv)


def _a2_body(qc_ref, wx_ref, bx_ref, wy_ref, by_ref, wa_ref, ba_ref,
             ref_ref, m4_ref, r1_ref, r2_ref, qb_ref, idx_ref, w_ref):
    qc = qc_ref[...]
    x64 = jnp.dot(qc, wx_ref[...], preferred_element_type=jnp.float32) + bx_ref[...]
    y64 = jnp.dot(qc, wy_ref[...], preferred_element_type=jnp.float32) + by_ref[...]
    a64 = jnp.dot(qc, wa_ref[...], preferred_element_type=jnp.float32) + ba_ref[...]
    m = jnp.maximum(a64, jnp.dot(a64, r1_ref[...], preferred_element_type=jnp.float32))
    m = jnp.maximum(m, jnp.dot(m, r2_ref[...], preferred_element_type=jnp.float32))
    e = jnp.exp(a64 - m)
    s = jnp.dot(e, m4_ref[...], preferred_element_type=jnp.float32)
    aw = e / s

    rx = ref_ref[:, 0:1]
    ry = ref_ref[:, 1:2]
    x = rx * W0 + x64 - 0.5
    y = ry * H0 + y64 - 0.5
    x0 = jnp.floor(x)
    y0 = jnp.floor(y)
    fx = x - x0
    fy = y - y0
    x1 = x0 + 1.0
    y1 = y0 + 1.0

    def corner(cx, cy, wgt):
        valid = ((cx >= 0.0) & (cx <= W0 - 1.0) & (cy >= 0.0) & (cy <= H0 - 1.0))
        cxc = jnp.clip(cx, 0.0, W0 - 1.0)
        cyc = jnp.clip(cy, 0.0, H0 - 1.0)
        flat = cyc * W0 + cxc + qb_ref[...]
        wv = wgt * aw * valid.astype(jnp.float32)
        return flat, wv

    fs, ws = zip(corner(x0, y0, (1.0 - fx) * (1.0 - fy)),
                 corner(x0, y1, (1.0 - fx) * fy),
                 corner(x1, y0, fx * (1.0 - fy)),
                 corner(x1, y1, fx * fy))
    idx_f = jnp.concatenate(
        [fs[c][:, p * 16:(p + 1) * 16] for p in range(POINTS) for c in range(4)], axis=1)
    w_f = jnp.concatenate(
        [ws[c][:, p * 16:(p + 1) * 16] for p in range(POINTS) for c in range(4)], axis=1)
    idx_ref[...] = idx_f.astype(jnp.int32)
    w_ref[...] = w_f


def _a2(qc, wxt, bx, wyt, by, wat, ba, refp, tq=512):
    nt = NQP // tq
    consts = dict(
        m4=jnp.asarray(_M4), r1=jnp.asarray(_R1), r2=jnp.asarray(_R2),
        qb=jnp.asarray(_QB).reshape(1, K64),
    )
    full = lambda shape: pl.BlockSpec(shape, lambda t: tuple(0 for _ in shape))
    return pl.pallas_call(
        _a2_body,
        grid=(nt,),
        in_specs=[
            pl.BlockSpec((tq, QUEUE * EMBED), lambda t: (t, 0)),
            full((QUEUE * EMBED, K64)), full((1, K64)),
            full((QUEUE * EMBED, K64)), full((1, K64)),
            full((QUEUE * EMBED, K64)), full((1, K64)),
            pl.BlockSpec((tq, 2), lambda t: (t, 0)),
            full((K64, K64)), full((K64, K64)), full((K64, K64)), full((1, K64)),
        ],
        out_specs=[
            pl.BlockSpec((tq, NTAP), lambda t: (t, 0)),
            pl.BlockSpec((tq, NTAP), lambda t: (t, 0)),
        ],
        out_shape=[
            jax.ShapeDtypeStruct((NQP, NTAP), jnp.int32),
            jax.ShapeDtypeStruct((NQP, NTAP), jnp.float32),
        ],
    )(qc, wxt, bx.reshape(1, K64), wyt, by.reshape(1, K64), wat,
      ba.reshape(1, K64), refp, consts["m4"], consts["r1"], consts["r2"], consts["qb"])


def _sc_gather(table, idx):
    B = idx.shape[0]
    info = plsc.get_sparse_core_info()
    nw = info.num_cores * info.num_subcores
    bw = B // nw
    chunk = 2048
    while bw % chunk:
        chunk //= 2
    niter = bw // chunk
    mesh = plsc.VectorSubcoreMesh(core_axis_name="c", subcore_axis_name="s")

    @functools.partial(
        pl.kernel, mesh=mesh,
        compiler_params=pltpu.CompilerParams(use_tc_tiling_on_sc=False),
        out_type=jax.ShapeDtypeStruct((B, HEAD_DIM), jnp.float32),
        scratch_types=[
            pltpu.VMEM((chunk,), jnp.int32),
            pltpu.VMEM((chunk, HEAD_DIM), jnp.float32),
            pltpu.SemaphoreType.DMA,
        ],
    )
    def k(table_hbm, idx_hbm, out_hbm, idx_v, rows_v, sem):
        wid = lax.axis_index("s") * info.num_cores + lax.axis_index("c")
        base0 = wid * bw

        def body(j, carry):
            base = base0 + j * chunk
            pltpu.sync_copy(idx_hbm.at[pl.ds(base, chunk)], idx_v)
            pltpu.async_copy(table_hbm.at[idx_v], rows_v, sem).wait()
            pltpu.sync_copy(rows_v, out_hbm.at[pl.ds(base, chunk)])
            return carry

        lax.fori_loop(0, niter, body, 0)

    return k(table, idx)


def _c_body(g_ref, w_ref, q_ref, wo_ref, bo_ref, rep_ref, out_ref):
    g = g_ref[...]
    w = w_ref[...]
    rep = rep_ref[...]
    slab = _NQH * HEAD_DIM
    o = None
    for s in range(POINTS * 4):
        wexp = jnp.dot(w[:, s * _NQH:(s + 1) * _NQH], rep,
                       preferred_element_type=jnp.float32)
        term = g[:, s * slab:(s + 1) * slab] * wexp
        o = term if o is None else o + term
    mean = 0.5 * (o[:, :EMBED] + o[:, EMBED:])
    out_ref[...] = (jnp.dot(mean, wo_ref[...], preferred_element_type=jnp.float32)
                    + bo_ref[...] + q_ref[...])


def _c(g2d, w2d, query_p, W_o, b_o, tq=128):
    nt = NQP // tq
    return pl.pallas_call(
        _c_body,
        grid=(nt,),
        in_specs=[
            pl.BlockSpec((tq, NTAP * HEAD_DIM), lambda t: (t, 0)),
            pl.BlockSpec((tq, NTAP), lambda t: (t, 0)),
            pl.BlockSpec((tq, EMBED), lambda t: (t, 0)),
            pl.BlockSpec((EMBED, EMBED), lambda t: (0, 0)),
            pl.BlockSpec((1, EMBED), lambda t: (0, 0)),
            pl.BlockSpec((_NQH, _NQH * HEAD_DIM), lambda t: (0, 0)),
        ],
        out_specs=pl.BlockSpec((tq, EMBED), lambda t: (t, 0)),
        out_shape=jax.ShapeDtypeStruct((NQP, EMBED), jnp.float32),
    )(g2d, w2d, query_p, W_o.T, b_o.reshape(1, EMBED), jnp.asarray(_REP))


def kernel(query, value, reference_points, spatial_shapes, W_so, b_so, W_aw, b_aw, W_v, b_v, W_o, b_o):
    pad = NQP - NQ
    value2 = value.reshape(QUEUE, NQ, EMBED)
    value_p = jnp.pad(value2, ((0, 0), (0, pad), (0, 0)))
    query_p = jnp.pad(query[0], ((0, pad), (0, 0)))
    refp = jnp.pad(reference_points.reshape(NQ, 2), ((0, pad), (0, 0)))
    qc = jnp.concatenate([value_p[0], query_p], axis=-1)

    table = _a1(value_p, W_v, b_v)
    wxt = W_so[jnp.asarray(_PX)].T
    wyt = W_so[jnp.asarray(_PY)].T
    bx = b_so[jnp.asarray(_PX)]
    by = b_so[jnp.asarray(_PY)]
    wat = W_aw[jnp.asarray(_PA)].T
    ba = b_aw[jnp.asarray(_PA)]
    idx, w = _a2(qc, wxt, bx, wyt, by, wat, ba, refp)

    g = _sc_gather(table.reshape(QUEUE * HEADS * NQP, HEAD_DIM),
                   idx.reshape(NQP * NTAP))
    out_p = _c(g.reshape(NQP, NTAP * HEAD_DIM), w, query_p, W_o, b_o)
    return out_p[None, :NQ, :]


# speedup vs baseline: 1362.4320x; 1.0335x over previous
import functools

import jax
import jax.numpy as jnp
import numpy as np
from jax import lax
from jax.experimental import pallas as pl
from jax.experimental.pallas import tpu as pltpu
from jax.experimental.pallas import tpu_sc as plsc

EMBED = 256
HEADS = 8
QUEUE = 2
POINTS = 4
H0 = 150
W0 = 150
NQ = H0 * W0
HEAD_DIM = EMBED // HEADS
NQP = 22528
NTAP = QUEUE * HEADS * POINTS * 4
K64 = QUEUE * HEADS * POINTS

def _perm_so(xy):
    rows = []
    for p in range(POINTS):
        for q in range(QUEUE):
            for h in range(HEADS):
                rows.append(((h * QUEUE + q) * POINTS + p) * 2 + xy)
    return np.array(rows, dtype=np.int32)

def _perm_aw():
    rows = []
    for p in range(POINTS):
        for q in range(QUEUE):
            for h in range(HEADS):
                rows.append((h * QUEUE + q) * POINTS + p)
    return np.array(rows, dtype=np.int32)

_PX = _perm_so(0)
_PY = _perm_so(1)
_PA = _perm_aw()
_NQH = QUEUE * HEADS
_M4 = np.zeros((K64, K64), np.float32)
_R1 = np.zeros((K64, K64), np.float32)
_R2 = np.zeros((K64, K64), np.float32)
for _i in range(K64):
    for _j in range(K64):
        if _i % _NQH == _j % _NQH:
            _M4[_i, _j] = 1.0
    _R1[_i, _i ^ 16] = 1.0
    _R2[_i, _i ^ 32] = 1.0
_QB = (np.arange(K64, dtype=np.float32) % _NQH) * NQP
_REP = np.zeros((_NQH, _NQH * HEAD_DIM), np.float32)
for _i in range(_NQH):
    _REP[_i, _i * HEAD_DIM:(_i + 1) * HEAD_DIM] = 1.0


def _a1_body(x_ref, wt_ref, b_ref, out_ref):
    x = x_ref[0]
    out_ref[0] = jnp.dot(x, wt_ref[0], preferred_element_type=jnp.float32) + b_ref[0]


def _a1(value_p, W_v, b_v, tq=512):
    nt = NQP // tq
    wvt = W_v.T.reshape(1, EMBED, HEADS, HEAD_DIM)
    wvt = jnp.transpose(wvt, (2, 0, 1, 3)).reshape(HEADS, EMBED, HEAD_DIM)
    bv = b_v.reshape(HEADS, 1, HEAD_DIM)
    return pl.pallas_call(
        _a1_body,
        grid=(QUEUE, nt, HEADS),
        in_specs=[
            pl.BlockSpec((1, tq, EMBED), lambda q, t, h: (q, t, 0)),
            pl.BlockSpec((1, EMBED, HEAD_DIM), lambda q, t, h: (h, 0, 0)),
            pl.BlockSpec((1, 1, HEAD_DIM), lambda q, t, h: (h, 0, 0)),
        ],
        out_specs=pl.BlockSpec((1, tq, HEAD_DIM), lambda q, t, h: (q * HEADS + h, t, 0)),
        out_shape=jax.ShapeDtypeStruct((QUEUE * HEADS, NQP, HEAD_DIM), jnp.float32),
    )(value_p, wvt.reshape(HEADS, EMBED, HEAD_DIM), bv)


def _a2_body(qc_ref, wx_ref, bx_ref, wy_ref, by_ref, wa_ref, ba_ref,
             ref_ref, m4_ref, r1_ref, r2_ref, qb_ref, idx_ref, w_ref):
    qc = qc_ref[...]
    x64 = jnp.dot(qc, wx_ref[...], preferred_element_type=jnp.float32) + bx_ref[...]
    y64 = jnp.dot(qc, wy_ref[...], preferred_element_type=jnp.float32) + by_ref[...]
    a64 = jnp.dot(qc, wa_ref[...], preferred_element_type=jnp.float32) + ba_ref[...]
    m = jnp.maximum(a64, jnp.dot(a64, r1_ref[...], preferred_element_type=jnp.float32))
    m = jnp.maximum(m, jnp.dot(m, r2_ref[...], preferred_element_type=jnp.float32))
    e = jnp.exp(a64 - m)
    s = jnp.dot(e, m4_ref[...], preferred_element_type=jnp.float32)
    aw = e / s

    rx = ref_ref[:, 0:1]
    ry = ref_ref[:, 1:2]
    x = rx * W0 + x64 - 0.5
    y = ry * H0 + y64 - 0.5
    x0 = jnp.floor(x)
    y0 = jnp.floor(y)
    fx = x - x0
    fy = y - y0
    x1 = x0 + 1.0
    y1 = y0 + 1.0

    def corner(cx, cy, wgt):
        valid = ((cx >= 0.0) & (cx <= W0 - 1.0) & (cy >= 0.0) & (cy <= H0 - 1.0))
        cxc = jnp.clip(cx, 0.0, W0 - 1.0)
        cyc = jnp.clip(cy, 0.0, H0 - 1.0)
        flat = cyc * W0 + cxc + qb_ref[...]
        wv = wgt * aw * valid.astype(jnp.float32)
        return flat, wv

    fs, ws = zip(corner(x0, y0, (1.0 - fx) * (1.0 - fy)),
                 corner(x0, y1, (1.0 - fx) * fy),
                 corner(x1, y0, fx * (1.0 - fy)),
                 corner(x1, y1, fx * fy))
    idx_f = jnp.concatenate(
        [fs[c][:, p * 16:(p + 1) * 16] for p in range(POINTS) for c in range(4)], axis=1)
    w_f = jnp.concatenate(
        [ws[c][:, p * 16:(p + 1) * 16] for p in range(POINTS) for c in range(4)], axis=1)
    idx_ref[...] = idx_f.astype(jnp.int32)
    w_ref[...] = w_f


def _a2(qc, wxt, bx, wyt, by, wat, ba, refp, tq=512):
    nt = NQP // tq
    consts = dict(
        m4=jnp.asarray(_M4), r1=jnp.asarray(_R1), r2=jnp.asarray(_R2),
        qb=jnp.asarray(_QB).reshape(1, K64),
    )
    full = lambda shape: pl.BlockSpec(shape, lambda t: tuple(0 for _ in shape))
    return pl.pallas_call(
        _a2_body,
        grid=(nt,),
        in_specs=[
            pl.BlockSpec((tq, QUEUE * EMBED), lambda t: (t, 0)),
            full((QUEUE * EMBED, K64)), full((1, K64)),
            full((QUEUE * EMBED, K64)), full((1, K64)),
            full((QUEUE * EMBED, K64)), full((1, K64)),
            pl.BlockSpec((tq, 2), lambda t: (t, 0)),
            full((K64, K64)), full((K64, K64)), full((K64, K64)), full((1, K64)),
        ],
        out_specs=[
            pl.BlockSpec((tq, NTAP), lambda t: (t, 0)),
            pl.BlockSpec((tq, NTAP), lambda t: (t, 0)),
        ],
        out_shape=[
            jax.ShapeDtypeStruct((NQP, NTAP), jnp.int32),
            jax.ShapeDtypeStruct((NQP, NTAP), jnp.float32),
        ],
    )(qc, wxt, bx.reshape(1, K64), wyt, by.reshape(1, K64), wat,
      ba.reshape(1, K64), refp, consts["m4"], consts["r1"], consts["r2"], consts["qb"])


def _sc_gather(table, idx):
    B = idx.shape[0]
    info = plsc.get_sparse_core_info()
    nw = info.num_cores * info.num_subcores
    bw = B // nw
    chunk = 1024
    while bw % (2 * chunk):
        chunk //= 2
    npair = bw // (2 * chunk)
    mesh = plsc.VectorSubcoreMesh(core_axis_name="c", subcore_axis_name="s")

    @functools.partial(
        pl.kernel, mesh=mesh,
        compiler_params=pltpu.CompilerParams(use_tc_tiling_on_sc=False),
        out_type=jax.ShapeDtypeStruct((B, HEAD_DIM), jnp.float32),
        scratch_types=[
            pltpu.VMEM((chunk,), jnp.int32),
            pltpu.VMEM((chunk,), jnp.int32),
            pltpu.VMEM((chunk, HEAD_DIM), jnp.float32),
            pltpu.VMEM((chunk, HEAD_DIM), jnp.float32),
            pltpu.SemaphoreType.DMA,
            pltpu.SemaphoreType.DMA,
            pltpu.SemaphoreType.DMA,
        ],
    )
    def k(table_hbm, idx_hbm, out_hbm, idx0, idx1, rows0, rows1, semi, semg, semw):
        wid = lax.axis_index("s") * info.num_cores + lax.axis_index("c")
        base0 = wid * bw
        last = base0 + bw - chunk
        pltpu.sync_copy(idx_hbm.at[pl.ds(base0, chunk)], idx0)

        def body(jj, carry):
            b0 = base0 + jj * (2 * chunk)
            b1 = b0 + chunk
            ci1 = pltpu.async_copy(idx_hbm.at[pl.ds(b1, chunk)], idx1, semi)
            pltpu.async_copy(table_hbm.at[idx0], rows0, semg).wait()
            cw0 = pltpu.async_copy(rows0, out_hbm.at[pl.ds(b0, chunk)], semw)
            ci1.wait()
            nb = jnp.minimum(b0 + 2 * chunk, last)
            ci0 = pltpu.async_copy(idx_hbm.at[pl.ds(nb, chunk)], idx0, semi)
            pltpu.async_copy(table_hbm.at[idx1], rows1, semg).wait()
            pltpu.sync_copy(rows1, out_hbm.at[pl.ds(b1, chunk)])
            cw0.wait()
            ci0.wait()
            return carry

        lax.fori_loop(0, npair, body, 0)

    return k(table, idx)


def _c_body(g_ref, w_ref, q_ref, wo_ref, bo_ref, rep_ref, out_ref):
    g = g_ref[...]
    w = w_ref[...]
    rep = rep_ref[...]
    slab = _NQH * HEAD_DIM
    o = None
    for s in range(POINTS * 4):
        wexp = jnp.dot(w[:, s * _NQH:(s + 1) * _NQH], rep,
                       preferred_element_type=jnp.float32)
        term = g[:, s * slab:(s + 1) * slab] * wexp
        o = term if o is None else o + term
    mean = 0.5 * (o[:, :EMBED] + o[:, EMBED:])
    out_ref[...] = (jnp.dot(mean, wo_ref[...], preferred_element_type=jnp.float32)
                    + bo_ref[...] + q_ref[...])


def _c(g2d, w2d, query_p, W_o, b_o, tq=128):
    nt = NQP // tq
    return pl.pallas_call(
        _c_body,
        grid=(nt,),
        in_specs=[
            pl.BlockSpec((tq, NTAP * HEAD_DIM), lambda t: (t, 0)),
            pl.BlockSpec((tq, NTAP), lambda t: (t, 0)),
            pl.BlockSpec((tq, EMBED), lambda t: (t, 0)),
            pl.BlockSpec((EMBED, EMBED), lambda t: (0, 0)),
            pl.BlockSpec((1, EMBED), lambda t: (0, 0)),
            pl.BlockSpec((_NQH, _NQH * HEAD_DIM), lambda t: (0, 0)),
        ],
        out_specs=pl.BlockSpec((tq, EMBED), lambda t: (t, 0)),
        out_shape=jax.ShapeDtypeStruct((NQP, EMBED), jnp.float32),
    )(g2d, w2d, query_p, W_o.T, b_o.reshape(1, EMBED), jnp.asarray(_REP))


def kernel(query, value, reference_points, spatial_shapes, W_so, b_so, W_aw, b_aw, W_v, b_v, W_o, b_o):
    pad = NQP - NQ
    value2 = value.reshape(QUEUE, NQ, EMBED)
    value_p = jnp.pad(value2, ((0, 0), (0, pad), (0, 0)))
    query_p = jnp.pad(query[0], ((0, pad), (0, 0)))
    refp = jnp.pad(reference_points.reshape(NQ, 2), ((0, pad), (0, 0)))
    qc = jnp.concatenate([value_p[0], query_p], axis=-1)

    table = _a1(value_p, W_v, b_v)
    wxt = W_so[jnp.asarray(_PX)].T
    wyt = W_so[jnp.asarray(_PY)].T
    bx = b_so[jnp.asarray(_PX)]
    by = b_so[jnp.asarray(_PY)]
    wat = W_aw[jnp.asarray(_PA)].T
    ba = b_aw[jnp.asarray(_PA)]
    idx, w = _a2(qc, wxt, bx, wyt, by, wat, ba, refp)

    g = _sc_gather(table.reshape(QUEUE * HEADS * NQP, HEAD_DIM),
                   idx.reshape(NQP * NTAP))
    out_p = _c(g.reshape(NQP, NTAP * HEAD_DIM), w, query_p, W_o, b_o)
    return out_p[None, :NQ, :]
